# Initial kernel scaffold; baseline (speedup 1.0000x reference)
import jax
import jax.numpy as jnp
from jax.experimental import pallas as pl


def kernel(inputs, embeddings):
    raise NotImplementedError("write your pallas kernel here")


# speedup vs baseline: 1.6389x; 1.6389x over previous
import jax
import jax.numpy as jnp
from jax import lax
from jax.experimental import pallas as pl
from jax.experimental.pallas import tpu as pltpu
from jax.experimental.pallas import tpu_sc as plsc

import functools


def _make_sc_kernel(B, S, W):
    info = plsc.get_sparse_core_info()
    NC, NS, L = info.num_cores, info.num_subcores, info.num_lanes
    NW = NC * NS
    mid = S // 2
    assert mid % NW == 0
    K = mid // NW
    assert K % L == 0

    mesh = plsc.VectorSubcoreMesh(core_axis_name="c", subcore_axis_name="s")

    @functools.partial(
        pl.kernel,
        out_type=jax.ShapeDtypeStruct((B * S, W), jnp.float32),
        mesh=mesh,
        scratch_types=[
            pltpu.VMEM((K, W), jnp.float32),
            pltpu.VMEM((K,), jnp.int32),
        ],
    )
    def k(emb_hbm, out_hbm, buf, idx):
        wid = lax.axis_index("s") * NC + lax.axis_index("c")
        d0 = wid * K

        pltpu.sync_copy(emb_hbm.at[pl.ds(d0, K)], buf)

        for b in range(B):
            pltpu.sync_copy(buf, out_hbm.at[pl.ds(b * S + mid + d0, K)])

            for c in range(K // L):
                base = b * S + mid - d0 - c * L
                idx[pl.ds(c * L, L)] = base - lax.iota(jnp.int32, L)
            pltpu.sync_copy(buf, out_hbm.at[idx])

        @pl.when(wid == NW - 1)
        def _():
            pltpu.sync_copy(emb_hbm.at[pl.ds(mid, 1)], buf.at[pl.ds(0, 1)])
            for b in range(B):
                pltpu.sync_copy(buf.at[pl.ds(0, 1)], out_hbm.at[pl.ds(b * S, 1)])

    return k


def kernel(inputs, embeddings):
    B, S, W = inputs.shape
    out = _make_sc_kernel(B, S, W)(embeddings)
    return out.reshape(B, S, W)


# speedup vs baseline: 1.6513x; 1.0076x over previous
import jax
import jax.numpy as jnp
from jax import lax
from jax.experimental import pallas as pl
from jax.experimental.pallas import tpu as pltpu
from jax.experimental.pallas import tpu_sc as plsc

import functools


def _make_sc_kernel(B, S, W):
    info = plsc.get_sparse_core_info()
    NC, NS, L = info.num_cores, info.num_subcores, info.num_lanes
    NW = NC * NS
    mid = S // 2
    assert mid % NW == 0
    K = mid // NW
    assert K % L == 0

    mesh = plsc.VectorSubcoreMesh(core_axis_name="c", subcore_axis_name="s")

    @functools.partial(
        pl.kernel,
        out_type=jax.ShapeDtypeStruct((B * S, W), jnp.float32),
        mesh=mesh,
        scratch_types=[
            pltpu.VMEM((K, W), jnp.float32),
            pltpu.VMEM((B, K), jnp.int32),
            pltpu.SemaphoreType.DMA,
        ],
    )
    def k(emb_hbm, out_hbm, buf, idx, sem):
        wid = lax.axis_index("s") * NC + lax.axis_index("c")
        d0 = wid * K

        pltpu.sync_copy(emb_hbm.at[pl.ds(d0, K)], buf)

        for b in range(B):
            for c in range(K // L):
                base = b * S + mid - d0 - c * L
                idx[b, pl.ds(c * L, L)] = base - lax.iota(jnp.int32, L)

        copies = []
        for b in range(B):
            copies.append(
                pltpu.async_copy(buf, out_hbm.at[pl.ds(b * S + mid + d0, K)], sem)
            )
            copies.append(pltpu.async_copy(buf, out_hbm.at[idx.at[b]], sem))
        for cp in copies:
            cp.wait()

        @pl.when(wid == NW - 1)
        def _():
            pltpu.sync_copy(emb_hbm.at[pl.ds(mid, 1)], buf.at[pl.ds(0, 1)])
            for b in range(B):
                pltpu.sync_copy(buf.at[pl.ds(0, 1)], out_hbm.at[pl.ds(b * S, 1)])

    return k


def kernel(inputs, embeddings):
    B, S, W = inputs.shape
    out = _make_sc_kernel(B, S, W)(embeddings)
    return out.reshape(B, S, W)
